# Initial kernel scaffold; baseline (speedup 1.0000x reference)
import jax
import jax.numpy as jnp
from jax.experimental import pallas as pl


def kernel(x, state, W_lin, b_lin, Wq, Wk, Wv, Wo, ln_g, ln_b, Wg1, bg1, Wg2, bg2, Wu, bu):
    raise NotImplementedError("write your pallas kernel here")


# speedup vs baseline: 84.8144x; 84.8144x over previous
import functools

import jax
import jax.numpy as jnp
import numpy as np
from jax.experimental import pallas as pl

B = 32
N = 512
DIN = 64
H = 64
TOPK = 32

_INT_MIN = np.int32(-2147483648)


def _kth_largest_threshold(s, k):
    i = jax.lax.bitcast_convert_type(s, jnp.int32)
    k32 = jnp.where(i < 0, _INT_MIN - i, i)
    u = k32 ^ _INT_MIN
    n_rows = s.shape[0]
    pref = jnp.zeros((n_rows, 1), jnp.int32)
    kk = jnp.full((n_rows, 1), float(k), jnp.float32)
    for b in range(31, -1, -1):
        bit = np.int32(-2147483648) if b == 31 else np.int32(1 << b)
        mask_hi = np.int32(-(1 << b))
        cand = pref | bit
        match = (u & mask_hi) == cand
        cnt = jnp.sum(match.astype(jnp.float32), axis=-1, keepdims=True)
        ge = cnt >= kk
        pref = jnp.where(ge, cand, pref)
        kk = jnp.where(ge, kk, kk - cnt)
    thresh = pref ^ _INT_MIN
    return k32, thresh


def _body(x_ref, st_ref, wlin_ref, blin_ref, wq_ref, wk_ref, wv_ref,
          wo_ref, lng_ref, lnb_ref, wg1_ref, bg1_ref, wg2_ref, bg2_ref,
          wu_ref, bu_ref, h_ref, aout_ref):
    f32 = jnp.float32
    dot = functools.partial(jnp.dot, preferred_element_type=f32)

    xb = x_ref[0]
    st = st_ref[0]
    xh = dot(xb, wlin_ref[...]) + blin_ref[...]
    ias = jnp.concatenate([xh, st], axis=-1)

    q = dot(ias, wq_ref[...])
    k_ = dot(ias, wk_ref[...])
    v = dot(ias, wv_ref[...])

    s = jax.lax.dot_general(q, k_, (((1,), (1,)), ((), ())),
                            preferred_element_type=f32) * (1.0 / 8.0)

    k32, thresh = _kth_largest_threshold(s, TOPK)
    mask = k32 >= thresh

    smax = jnp.max(jnp.where(mask, s, -jnp.inf), axis=-1, keepdims=True)
    e = jnp.where(mask, jnp.exp(s - smax), 0.0)
    denom = jnp.sum(e, axis=-1, keepdims=True)
    attn = e / denom

    ctx = dot(attn, v)
    pre = ias + dot(ctx, wo_ref[...])
    mu = jnp.mean(pre, axis=-1, keepdims=True)
    var = jnp.mean((pre - mu) * (pre - mu), axis=-1, keepdims=True)
    xx = (pre - mu) * jax.lax.rsqrt(var + 1e-6) * lng_ref[...] + lnb_ref[...]

    deg = jnp.sum(attn, axis=-1, keepdims=True) + 1.0
    dinv = jax.lax.rsqrt(deg)

    def gcn(xf, w_ref, b_ref):
        xw = dot(xf, w_ref[...])
        agg = dot(attn, dinv * xw)
        return dinv * agg + (dinv * dinv) * xw + b_ref[...]

    z = jax.nn.sigmoid(gcn(xx, wg1_ref, bg1_ref))
    r = jax.nn.sigmoid(gcn(xx, wg2_ref, bg2_ref))
    cand = jnp.concatenate([xx, z * st], axis=-1)
    hc = jnp.tanh(gcn(cand, wu_ref, bu_ref))
    h_ref[0] = r * st + (1.0 - r) * hc
    aout_ref[0] = attn.T


def kernel(x, state, W_lin, b_lin, Wq, Wk, Wv, Wo, ln_g, ln_b,
           Wg1, bg1, Wg2, bg2, Wu, bu):
    row = lambda a: a.reshape(1, -1)
    full = lambda shp: pl.BlockSpec(shp, lambda b: (0,) * len(shp))
    per_b = lambda shp: pl.BlockSpec((1,) + shp, lambda b: (b, 0, 0))

    h, a_out = pl.pallas_call(
        _body,
        grid=(B,),
        in_specs=[
            per_b((N, DIN)),
            per_b((N, H)),
            full((DIN, H)),
            full((1, H)),
            full((2 * H, H)),
            full((2 * H, H)),
            full((2 * H, H)),
            full((H, 2 * H)),
            full((1, 2 * H)),
            full((1, 2 * H)),
            full((2 * H, H)),
            full((1, H)),
            full((2 * H, H)),
            full((1, H)),
            full((3 * H, H)),
            full((1, H)),
        ],
        out_specs=[
            per_b((N, H)),
            per_b((N, N)),
        ],
        out_shape=[
            jax.ShapeDtypeStruct((B, N, H), jnp.float32),
            jax.ShapeDtypeStruct((B, N, N), jnp.float32),
        ],
    )(x, state, W_lin, row(b_lin), Wq, Wk, Wv, Wo, row(ln_g), row(ln_b),
      Wg1, row(bg1), Wg2, row(bg2), Wu, row(bu))
    return h, a_out


# speedup vs baseline: 101.7159x; 1.1993x over previous
import functools

import jax
import jax.numpy as jnp
import numpy as np
from jax.experimental import pallas as pl

B = 32
N = 512
DIN = 64
H = 64
TOPK = 32

_INT_MIN = np.int32(-2147483648)


def _kth_largest_threshold(s, k):
    i = jax.lax.bitcast_convert_type(s, jnp.int32)
    k32 = jnp.where(i < 0, _INT_MIN - i, i)
    n_rows = s.shape[0]
    t = jnp.zeros((n_rows, 1), jnp.int32)
    for b in range(31, -1, -1):
        bit = np.int32(-2147483648) if b == 31 else np.int32(1 << b)
        cand = t | bit
        ge = k32 >= (cand ^ _INT_MIN)
        cnt = jnp.sum(ge.astype(jnp.float32), axis=-1, keepdims=True)
        t = jnp.where(cnt >= float(k), cand, t)
    thresh = t ^ _INT_MIN
    return k32, thresh


def _body(x_ref, st_ref, wlin_ref, blin_ref, wq_ref, wk_ref, wv_ref,
          wo_ref, lng_ref, lnb_ref, wg1_ref, bg1_ref, wg2_ref, bg2_ref,
          wu_ref, bu_ref, h_ref, aout_ref):
    f32 = jnp.float32
    dot = functools.partial(jnp.dot, preferred_element_type=f32)

    xb = x_ref[0]
    st = st_ref[0]
    xh = dot(xb, wlin_ref[...]) + blin_ref[...]
    ias = jnp.concatenate([xh, st], axis=-1)

    q = dot(ias, wq_ref[...])
    k_ = dot(ias, wk_ref[...])
    v = dot(ias, wv_ref[...])

    s = jax.lax.dot_general(q, k_, (((1,), (1,)), ((), ())),
                            preferred_element_type=f32) * (1.0 / 8.0)

    k32, thresh = _kth_largest_threshold(s, TOPK)
    mask = k32 >= thresh

    smax = jnp.max(s, axis=-1, keepdims=True)
    e = jnp.where(mask, jnp.exp(s - smax), 0.0)
    denom = jnp.sum(e, axis=-1, keepdims=True)
    attn = e / denom

    ctx = dot(attn, v)
    pre = ias + dot(ctx, wo_ref[...])
    mu = jnp.mean(pre, axis=-1, keepdims=True)
    var = jnp.mean((pre - mu) * (pre - mu), axis=-1, keepdims=True)
    xx = (pre - mu) * jax.lax.rsqrt(var + 1e-6) * lng_ref[...] + lnb_ref[...]

    deg = jnp.sum(attn, axis=-1, keepdims=True) + 1.0
    dinv = jax.lax.rsqrt(deg)

    def gcn(xf, w_ref, b_ref):
        xw = dot(xf, w_ref[...])
        agg = dot(attn, dinv * xw)
        return dinv * agg + (dinv * dinv) * xw + b_ref[...]

    z = jax.nn.sigmoid(gcn(xx, wg1_ref, bg1_ref))
    r = jax.nn.sigmoid(gcn(xx, wg2_ref, bg2_ref))
    cand = jnp.concatenate([xx, z * st], axis=-1)
    hc = jnp.tanh(gcn(cand, wu_ref, bu_ref))
    h_ref[0] = r * st + (1.0 - r) * hc
    aout_ref[0] = attn.T


def kernel(x, state, W_lin, b_lin, Wq, Wk, Wv, Wo, ln_g, ln_b,
           Wg1, bg1, Wg2, bg2, Wu, bu):
    row = lambda a: a.reshape(1, -1)
    full = lambda shp: pl.BlockSpec(shp, lambda b: (0,) * len(shp))
    per_b = lambda shp: pl.BlockSpec((1,) + shp, lambda b: (b, 0, 0))

    h, a_out = pl.pallas_call(
        _body,
        grid=(B,),
        in_specs=[
            per_b((N, DIN)),
            per_b((N, H)),
            full((DIN, H)),
            full((1, H)),
            full((2 * H, H)),
            full((2 * H, H)),
            full((2 * H, H)),
            full((H, 2 * H)),
            full((1, 2 * H)),
            full((1, 2 * H)),
            full((2 * H, H)),
            full((1, H)),
            full((2 * H, H)),
            full((1, H)),
            full((3 * H, H)),
            full((1, H)),
        ],
        out_specs=[
            per_b((N, H)),
            per_b((N, N)),
        ],
        out_shape=[
            jax.ShapeDtypeStruct((B, N, H), jnp.float32),
            jax.ShapeDtypeStruct((B, N, N), jnp.float32),
        ],
    )(x, state, W_lin, row(b_lin), Wq, Wk, Wv, Wo, row(ln_g), row(ln_b),
      Wg1, row(bg1), Wg2, row(bg2), Wu, row(bu))
    return h, a_out


# speedup vs baseline: 113.5393x; 1.1162x over previous
import functools

import jax
import jax.numpy as jnp
import numpy as np
from jax.experimental import pallas as pl

B = 32
N = 512
DIN = 64
H = 64
TOPK = 32

_INT_MIN = np.int32(-2147483648)


_RADIX_PASSES = 26


def _kth_largest_threshold(s, k):
    i = jax.lax.bitcast_convert_type(s, jnp.int32)
    k32 = jnp.where(i < 0, _INT_MIN - i, i)
    n_rows = s.shape[0]
    t = jnp.zeros((n_rows, 1), jnp.int32)
    for b in range(31, 31 - _RADIX_PASSES, -1):
        bit = np.int32(-2147483648) if b == 31 else np.int32(1 << b)
        cand = t | bit
        ge = k32 >= (cand ^ _INT_MIN)
        cnt = jnp.sum(ge.astype(jnp.float32), axis=-1, keepdims=True)
        t = jnp.where(cnt >= float(k), cand, t)
    thresh = t ^ _INT_MIN
    return k32, thresh


def _body(x_ref, st_ref, wlin_ref, blin_ref, wq_ref, wk_ref, wv_ref,
          wo_ref, lng_ref, lnb_ref, wg1_ref, bg1_ref, wg2_ref, bg2_ref,
          wu_ref, bu_ref, h_ref, aout_ref):
    f32 = jnp.float32
    dot = functools.partial(jnp.dot, preferred_element_type=f32)

    xb = x_ref[0]
    st = st_ref[0]
    xh = dot(xb, wlin_ref[...]) + blin_ref[...]
    ias = jnp.concatenate([xh, st], axis=-1)

    q = dot(ias, wq_ref[...])
    k_ = dot(ias, wk_ref[...])
    v = dot(ias, wv_ref[...])

    s = jax.lax.dot_general(q, k_, (((1,), (1,)), ((), ())),
                            preferred_element_type=f32) * (1.0 / 8.0)

    k32, thresh = _kth_largest_threshold(s, TOPK)
    mask = k32 >= thresh

    smax = jnp.max(s, axis=-1, keepdims=True)
    e = jnp.where(mask, jnp.exp(s - smax), 0.0)
    denom = jnp.sum(e, axis=-1, keepdims=True)
    attn = e / denom

    ctx = dot(attn, v)
    pre = ias + dot(ctx, wo_ref[...])
    mu = jnp.mean(pre, axis=-1, keepdims=True)
    var = jnp.mean((pre - mu) * (pre - mu), axis=-1, keepdims=True)
    xx = (pre - mu) * jax.lax.rsqrt(var + 1e-6) * lng_ref[...] + lnb_ref[...]

    def gcn(xf, w_ref, b_ref):
        xw = dot(xf, w_ref[...])
        return 0.5 * (dot(attn, xw) + xw) + b_ref[...]

    z = jax.nn.sigmoid(gcn(xx, wg1_ref, bg1_ref))
    r = jax.nn.sigmoid(gcn(xx, wg2_ref, bg2_ref))
    cand = jnp.concatenate([xx, z * st], axis=-1)
    hc = jnp.tanh(gcn(cand, wu_ref, bu_ref))
    h_ref[0] = r * st + (1.0 - r) * hc
    aout_ref[0] = attn.T


def kernel(x, state, W_lin, b_lin, Wq, Wk, Wv, Wo, ln_g, ln_b,
           Wg1, bg1, Wg2, bg2, Wu, bu):
    row = lambda a: a.reshape(1, -1)
    full = lambda shp: pl.BlockSpec(shp, lambda b: (0,) * len(shp))
    per_b = lambda shp: pl.BlockSpec((1,) + shp, lambda b: (b, 0, 0))

    h, a_out = pl.pallas_call(
        _body,
        grid=(B,),
        in_specs=[
            per_b((N, DIN)),
            per_b((N, H)),
            full((DIN, H)),
            full((1, H)),
            full((2 * H, H)),
            full((2 * H, H)),
            full((2 * H, H)),
            full((H, 2 * H)),
            full((1, 2 * H)),
            full((1, 2 * H)),
            full((2 * H, H)),
            full((1, H)),
            full((2 * H, H)),
            full((1, H)),
            full((3 * H, H)),
            full((1, H)),
        ],
        out_specs=[
            per_b((N, H)),
            per_b((N, N)),
        ],
        out_shape=[
            jax.ShapeDtypeStruct((B, N, H), jnp.float32),
            jax.ShapeDtypeStruct((B, N, N), jnp.float32),
        ],
    )(x, state, W_lin, row(b_lin), Wq, Wk, Wv, Wo, row(ln_g), row(ln_b),
      Wg1, row(bg1), Wg2, row(bg2), Wu, row(bu))
    return h, a_out


# speedup vs baseline: 138.7941x; 1.2224x over previous
import functools

import jax
import jax.numpy as jnp
import numpy as np
from jax.experimental import pallas as pl

B = 32
N = 512
DIN = 64
H = 64
TOPK = 32

_INT_MIN = np.int32(-2147483648)


_RADIX_PASSES = 26


_BPP = 2


def _body(x_ref, st_ref, wlin_ref, blin_ref, wq_ref, wk_ref, wv_ref,
          wo_ref, lng_ref, lnb_ref, wg1_ref, bg1_ref, wg2_ref, bg2_ref,
          wu_ref, bu_ref, h_ref, aout_ref):
    f32 = jnp.float32
    dot = functools.partial(jnp.dot, preferred_element_type=f32)

    ias_l, st_l, v_l, s_l, k32_l = [], [], [], [], []
    for j in range(_BPP):
        xb = x_ref[j]
        st = st_ref[j]
        xh = dot(xb, wlin_ref[...]) + blin_ref[...]
        ias = jnp.concatenate([xh, st], axis=-1)
        q = dot(ias, wq_ref[...])
        k_ = dot(ias, wk_ref[...])
        v = dot(ias, wv_ref[...])
        s = jax.lax.dot_general(q, k_, (((1,), (1,)), ((), ())),
                                preferred_element_type=f32) * (1.0 / 8.0)
        i = jax.lax.bitcast_convert_type(s, jnp.int32)
        k32 = jnp.where(i < 0, _INT_MIN - i, i)
        ias_l.append(ias); st_l.append(st); v_l.append(v)
        s_l.append(s); k32_l.append(k32)

    kf = float(TOPK)

    def count_ge(j, cand_s):
        ge = k32_l[j] >= cand_s
        return jnp.sum(ge.astype(f32), axis=-1, keepdims=True)

    ts_l = [jnp.where(count_ge(j, jnp.int32(0)) >= kf, jnp.int32(0), _INT_MIN)
            for j in range(_BPP)]
    for b in range(30, 31 - _RADIX_PASSES, -1):
        bit = np.int32(1 << b)
        for j in range(_BPP):
            cand_s = ts_l[j] | bit
            ts_l[j] = jnp.where(count_ge(j, cand_s) >= kf, cand_s, ts_l[j])

    for j in range(_BPP):
        s, k32, ias, st, v = s_l[j], k32_l[j], ias_l[j], st_l[j], v_l[j]
        mask = k32 >= ts_l[j]
        smax = jnp.max(s, axis=-1, keepdims=True)
        e = jnp.where(mask, jnp.exp(s - smax), 0.0)
        denom = jnp.sum(e, axis=-1, keepdims=True)
        attn = e / denom

        ctx = dot(attn, v)
        pre = ias + dot(ctx, wo_ref[...])
        mu = jnp.mean(pre, axis=-1, keepdims=True)
        var = jnp.mean((pre - mu) * (pre - mu), axis=-1, keepdims=True)
        xx = (pre - mu) * jax.lax.rsqrt(var + 1e-6) * lng_ref[...] + lnb_ref[...]

        def gcn(xf, w_ref, b_ref):
            xw = dot(xf, w_ref[...])
            return 0.5 * (dot(attn, xw) + xw) + b_ref[...]

        z = jax.nn.sigmoid(gcn(xx, wg1_ref, bg1_ref))
        r = jax.nn.sigmoid(gcn(xx, wg2_ref, bg2_ref))
        cand = jnp.concatenate([xx, z * st], axis=-1)
        hc = jnp.tanh(gcn(cand, wu_ref, bu_ref))
        h_ref[j] = r * st + (1.0 - r) * hc
        aout_ref[j] = attn.T


def kernel(x, state, W_lin, b_lin, Wq, Wk, Wv, Wo, ln_g, ln_b,
           Wg1, bg1, Wg2, bg2, Wu, bu):
    row = lambda a: a.reshape(1, -1)
    full = lambda shp: pl.BlockSpec(shp, lambda b: (0,) * len(shp))
    per_b = lambda shp: pl.BlockSpec((_BPP,) + shp, lambda b: (b, 0, 0))

    h, a_out = pl.pallas_call(
        _body,
        grid=(B // _BPP,),
        in_specs=[
            per_b((N, DIN)),
            per_b((N, H)),
            full((DIN, H)),
            full((1, H)),
            full((2 * H, H)),
            full((2 * H, H)),
            full((2 * H, H)),
            full((H, 2 * H)),
            full((1, 2 * H)),
            full((1, 2 * H)),
            full((2 * H, H)),
            full((1, H)),
            full((2 * H, H)),
            full((1, H)),
            full((3 * H, H)),
            full((1, H)),
        ],
        out_specs=[
            per_b((N, H)),
            per_b((N, N)),
        ],
        out_shape=[
            jax.ShapeDtypeStruct((B, N, H), jnp.float32),
            jax.ShapeDtypeStruct((B, N, N), jnp.float32),
        ],
    )(x, state, W_lin, row(b_lin), Wq, Wk, Wv, Wo, row(ln_g), row(ln_b),
      Wg1, row(bg1), Wg2, row(bg2), Wu, row(bu))
    return h, a_out


# speedup vs baseline: 143.9961x; 1.0375x over previous
import functools

import jax
import jax.numpy as jnp
import numpy as np
from jax.experimental import pallas as pl

B = 32
N = 512
DIN = 64
H = 64
TOPK = 32

_INT_MIN = np.int32(-2147483648)

_RADIX_PASSES = 26

_BPP = 2


def _body(x_ref, st_ref, wlin_ref, blin_ref, wq_ref, wk_ref, wv_ref,
          wo_ref, lng_ref, lnb_ref, wg1_ref, bg1_ref, wg2_ref, bg2_ref,
          wu_ref, bu_ref, h_ref, aout_ref):
    f32 = jnp.float32
    dot = functools.partial(jnp.dot, preferred_element_type=f32)

    ias_l, st_l, v_l, s_l, k32_l = [], [], [], [], []
    for j in range(_BPP):
        xbT = x_ref[j].T
        stT = st_ref[j].T
        xhT = dot(wlin_ref[...], xbT) + blin_ref[...]
        iasT = jnp.concatenate([xhT, stT], axis=0)
        qT = dot(wq_ref[...], iasT)
        kT = dot(wk_ref[...], iasT)
        vT = dot(wv_ref[...], iasT)
        sT = dot(kT.T, qT) * (1.0 / 8.0)
        i = jax.lax.bitcast_convert_type(sT, jnp.int32)
        k32 = jnp.where(i < 0, _INT_MIN - i, i)
        ias_l.append(iasT); st_l.append(stT); v_l.append(vT)
        s_l.append(sT); k32_l.append(k32)

    kf = float(TOPK)

    ones_row = jnp.ones((1, N), f32)

    def count_ge(j, cand_s):
        ge = k32_l[j] >= cand_s
        return jnp.dot(ones_row, ge.astype(f32), preferred_element_type=f32)

    ts_l = [jnp.where(count_ge(j, jnp.int32(0)) >= kf, jnp.int32(0), _INT_MIN)
            for j in range(_BPP)]
    for b in range(30, 31 - _RADIX_PASSES, -1):
        bit = np.int32(1 << b)
        for j in range(_BPP):
            cand_s = ts_l[j] | bit
            ts_l[j] = jnp.where(count_ge(j, cand_s) >= kf, cand_s, ts_l[j])

    for j in range(_BPP):
        sT, k32, iasT, stT, vT = s_l[j], k32_l[j], ias_l[j], st_l[j], v_l[j]
        mask = k32 >= ts_l[j]
        smax = jnp.max(sT, axis=0, keepdims=True)
        e = jnp.where(mask, jnp.exp(sT - smax), 0.0)
        denom = jnp.sum(e, axis=0, keepdims=True)
        attnT = e / denom

        ctxT = dot(vT, attnT)
        preT = iasT + dot(wo_ref[...], ctxT)
        mu = jnp.mean(preT, axis=0, keepdims=True)
        dev = preT - mu
        var = jnp.mean(dev * dev, axis=0, keepdims=True)
        xxT = dev * jax.lax.rsqrt(var + 1e-6) * lng_ref[...] + lnb_ref[...]

        def gcn(xfT, w_ref, b_ref):
            xwT = dot(w_ref[...], xfT)
            return 0.5 * (dot(xwT, attnT) + xwT) + b_ref[...]

        z = jax.nn.sigmoid(gcn(xxT, wg1_ref, bg1_ref))
        r = jax.nn.sigmoid(gcn(xxT, wg2_ref, bg2_ref))
        candT = jnp.concatenate([xxT, z * stT], axis=0)
        hcT = jnp.tanh(gcn(candT, wu_ref, bu_ref))
        h_ref[j] = (r * stT + (1.0 - r) * hcT).T
        aout_ref[j] = attnT


def kernel(x, state, W_lin, b_lin, Wq, Wk, Wv, Wo, ln_g, ln_b,
           Wg1, bg1, Wg2, bg2, Wu, bu):
    col = lambda a: a.reshape(-1, 1)
    full = lambda shp: pl.BlockSpec(shp, lambda b: (0,) * len(shp))
    per_b = lambda shp: pl.BlockSpec((_BPP,) + shp, lambda b: (b, 0, 0))

    h, a_out = pl.pallas_call(
        _body,
        grid=(B // _BPP,),
        in_specs=[
            per_b((N, DIN)),
            per_b((N, H)),
            full((H, DIN)),
            full((H, 1)),
            full((H, 2 * H)),
            full((H, 2 * H)),
            full((H, 2 * H)),
            full((2 * H, H)),
            full((2 * H, 1)),
            full((2 * H, 1)),
            full((H, 2 * H)),
            full((H, 1)),
            full((H, 2 * H)),
            full((H, 1)),
            full((H, 3 * H)),
            full((H, 1)),
        ],
        out_specs=[
            per_b((N, H)),
            per_b((N, N)),
        ],
        out_shape=[
            jax.ShapeDtypeStruct((B, N, H), jnp.float32),
            jax.ShapeDtypeStruct((B, N, N), jnp.float32),
        ],
    )(x, state, W_lin.T, col(b_lin), Wq.T, Wk.T, Wv.T, Wo.T, col(ln_g),
      col(ln_b), Wg1.T, col(bg1), Wg2.T, col(bg2), Wu.T, col(bu))
    return h, a_out


# speedup vs baseline: 164.2632x; 1.1407x over previous
import functools

import jax
import jax.numpy as jnp
import numpy as np
from jax.experimental import pallas as pl

B = 32
N = 512
DIN = 64
H = 64
TOPK = 32

_INT_MIN = np.int32(-2147483648)

_RADIX_PASSES = 26

_BPP = 4


def _body(x_ref, st_ref, wlin_ref, blin_ref, wq_ref, wk_ref, wv_ref,
          wo_ref, lng_ref, lnb_ref, wg1_ref, bg1_ref, wg2_ref, bg2_ref,
          wu_ref, bu_ref, h_ref, aout_ref):
    f32 = jnp.float32
    dot = functools.partial(jnp.dot, preferred_element_type=f32)

    ias_l, st_l, v_l, s_l, k32_l = [], [], [], [], []
    for j in range(_BPP):
        xbT = x_ref[j].T
        stT = st_ref[j].T
        xhT = dot(wlin_ref[...], xbT) + blin_ref[...]
        iasT = jnp.concatenate([xhT, stT], axis=0)
        qT = dot(wq_ref[...], iasT)
        kT = dot(wk_ref[...], iasT)
        vT = dot(wv_ref[...], iasT)
        sT = dot(kT.T, qT) * (1.0 / 8.0)
        i = jax.lax.bitcast_convert_type(sT, jnp.int32)
        k32 = jnp.where(i < 0, _INT_MIN - i, i)
        ias_l.append(iasT); st_l.append(stT); v_l.append(vT)
        s_l.append(sT); k32_l.append(k32)

    kf = float(TOPK)

    ones_row = jnp.ones((1, N), f32)

    def count_ge(j, cand_s):
        ge = k32_l[j] >= cand_s
        return jnp.dot(ones_row, ge.astype(f32), preferred_element_type=f32)

    ts_l = [jnp.where(count_ge(j, jnp.int32(0)) >= kf, jnp.int32(0), _INT_MIN)
            for j in range(_BPP)]
    for b in range(30, 31 - _RADIX_PASSES, -1):
        bit = np.int32(1 << b)
        for j in range(_BPP):
            cand_s = ts_l[j] | bit
            ts_l[j] = jnp.where(count_ge(j, cand_s) >= kf, cand_s, ts_l[j])

    for j in range(_BPP):
        sT, k32, iasT, stT, vT = s_l[j], k32_l[j], ias_l[j], st_l[j], v_l[j]
        mask = k32 >= ts_l[j]
        smax = jnp.max(sT, axis=0, keepdims=True)
        e = jnp.where(mask, jnp.exp(sT - smax), 0.0)
        denom = jnp.sum(e, axis=0, keepdims=True)
        attnT = e / denom

        ctxT = dot(vT, attnT)
        preT = iasT + dot(wo_ref[...], ctxT)
        mu = jnp.mean(preT, axis=0, keepdims=True)
        dev = preT - mu
        var = jnp.mean(dev * dev, axis=0, keepdims=True)
        xxT = dev * jax.lax.rsqrt(var + 1e-6) * lng_ref[...] + lnb_ref[...]

        def gcn(xfT, w_ref, b_ref):
            xwT = dot(w_ref[...], xfT)
            return 0.5 * (dot(xwT, attnT) + xwT) + b_ref[...]

        z = jax.nn.sigmoid(gcn(xxT, wg1_ref, bg1_ref))
        r = jax.nn.sigmoid(gcn(xxT, wg2_ref, bg2_ref))
        candT = jnp.concatenate([xxT, z * stT], axis=0)
        hcT = jnp.tanh(gcn(candT, wu_ref, bu_ref))
        h_ref[j] = (r * stT + (1.0 - r) * hcT).T
        aout_ref[j] = attnT


def kernel(x, state, W_lin, b_lin, Wq, Wk, Wv, Wo, ln_g, ln_b,
           Wg1, bg1, Wg2, bg2, Wu, bu):
    col = lambda a: a.reshape(-1, 1)
    full = lambda shp: pl.BlockSpec(shp, lambda b: (0,) * len(shp))
    per_b = lambda shp: pl.BlockSpec((_BPP,) + shp, lambda b: (b, 0, 0))

    h, a_out = pl.pallas_call(
        _body,
        grid=(B // _BPP,),
        in_specs=[
            per_b((N, DIN)),
            per_b((N, H)),
            full((H, DIN)),
            full((H, 1)),
            full((H, 2 * H)),
            full((H, 2 * H)),
            full((H, 2 * H)),
            full((2 * H, H)),
            full((2 * H, 1)),
            full((2 * H, 1)),
            full((H, 2 * H)),
            full((H, 1)),
            full((H, 2 * H)),
            full((H, 1)),
            full((H, 3 * H)),
            full((H, 1)),
        ],
        out_specs=[
            per_b((N, H)),
            per_b((N, N)),
        ],
        out_shape=[
            jax.ShapeDtypeStruct((B, N, H), jnp.float32),
            jax.ShapeDtypeStruct((B, N, N), jnp.float32),
        ],
    )(x, state, W_lin.T, col(b_lin), Wq.T, Wk.T, Wv.T, Wo.T, col(ln_g),
      col(ln_b), Wg1.T, col(bg1), Wg2.T, col(bg2), Wu.T, col(bu))
    return h, a_out
